# baseline (speedup 1.0000x reference)
import functools

import jax
import jax.numpy as jnp
from jax import lax
from jax.experimental import pallas as pl
from jax.experimental.pallas import tpu as pltpu
from jax.experimental.pallas import tpu_sc as plsc

N = 10000
E = 320000
F_IN = 128
H = 64
OUT = 128
G = 16

NC, NS, L = 2, 16, 16
NW = NC * NS
PT = 640
NPAD = NS * PT
CH = 128
EC = 80
PE = NW * EC * CH
NBUF = 4

_f32 = jnp.float32
_mesh = plsc.VectorSubcoreMesh(core_axis_name="c", subcore_axis_name="s")
_sc_params = pltpu.CompilerParams(use_tc_tiling_on_sc=False)


@functools.partial(
    pl.kernel,
    out_type=jax.ShapeDtypeStruct((NC, NPAD), _f32),
    mesh=_mesh,
    compiler_params=_sc_params,
    scratch_types=[
        pltpu.VMEM((EC, CH), jnp.int32),
        pltpu.VMEM((CH,), _f32),
        pltpu.VMEM((PT,), _f32),
        pltpu.VMEM_SHARED((NPAD,), _f32),
    ],
)
def _deg_kernel(dst_hbm, out_hbm, idx_d, ones_v, zbuf, acc):
    c = lax.axis_index("c")
    s = lax.axis_index("s")
    wid = c * NS + s
    pltpu.sync_copy(dst_hbm.at[wid], idx_d)
    for i in range(CH // L):
        ones_v[pl.ds(i * L, L)] = jnp.ones((L,), _f32)

    @pl.loop(0, PT // L)
    def _(i):
        zbuf[pl.ds(i * L, L)] = jnp.zeros((L,), _f32)

    pltpu.sync_copy(zbuf, acc.at[pl.ds(s * PT, PT)])
    plsc.subcore_barrier()

    @pl.loop(0, EC)
    def _(j):
        pltpu.sync_copy(ones_v, acc.at[idx_d.at[j]], add=True)

    plsc.subcore_barrier()
    pltpu.sync_copy(acc.at[pl.ds(s * PT, PT)], out_hbm.at[c, pl.ds(s * PT, PT)])


@functools.partial(
    pl.kernel,
    out_type=jax.ShapeDtypeStruct((NC, NPAD, H), _f32),
    mesh=_mesh,
    compiler_params=_sc_params,
    scratch_types=(
        [pltpu.VMEM((EC, CH), jnp.int32),
         pltpu.VMEM((EC, CH), jnp.int32)]
        + [pltpu.VMEM((CH, H), _f32) for _ in range(NBUF)]
        + [pltpu.VMEM_SHARED((NPAD, H), _f32)]
        + [pltpu.SemaphoreType.DMA for _ in range(2 * NBUF)]
    ),
)
def _mp_kernel(y_hbm, src_hbm, dst_hbm, out_hbm, idx_s, idx_d, *rest):
    bufs = rest[:NBUF]
    acc = rest[NBUF]
    gsems = rest[NBUF + 1:2 * NBUF + 1]
    ssems = rest[2 * NBUF + 1:]
    c = lax.axis_index("c")
    s = lax.axis_index("s")
    wid = c * NS + s
    pltpu.sync_copy(src_hbm.at[wid], idx_s)
    pltpu.sync_copy(dst_hbm.at[wid], idx_d)

    @pl.when(c == 0)
    def _():
        pltpu.sync_copy(y_hbm.at[pl.ds(s * PT, PT)], acc.at[pl.ds(s * PT, PT)])

    @pl.when(c == 1)
    def _():
        @pl.loop(0, CH)
        def _(i):
            for jj in range(H // L):
                bufs[0][i, pl.ds(jj * L, L)] = jnp.zeros((L,), _f32)

        @pl.loop(0, PT // CH)
        def _(k):
            pltpu.sync_copy(bufs[0], acc.at[pl.ds(s * PT + k * CH, CH)])

    plsc.subcore_barrier()

    def _gather(ch, b):
        return pltpu.make_async_copy(y_hbm.at[idx_s.at[ch]], bufs[b], gsems[b])

    def _scatter_start(ch, b):
        pltpu.async_copy(bufs[b], acc.at[idx_d.at[ch]], ssems[b], add=True)

    def _scatter_wait(b):
        pltpu.make_async_copy(bufs[b], acc.at[idx_d.at[0]], ssems[b]).wait()

    @pl.loop(0, EC)
    def _(j):
        pltpu.async_copy(y_hbm.at[idx_s.at[j]], bufs[0], gsems[0]).wait()

    plsc.subcore_barrier()
    pltpu.sync_copy(acc.at[pl.ds(s * PT, PT)], out_hbm.at[c, pl.ds(s * PT, PT)])


_BT = 2048


def _tc1_body(d0_ref, d1_ref, x_ref, w_ref, dis_ref, y_ref):
    i = pl.program_id(0)
    deg = d0_ref[...] + d1_ref[...] + 1.0
    rows = lax.broadcasted_iota(jnp.int32, (_BT, 1), 0) + i * _BT
    dis = jnp.where(rows < N, lax.rsqrt(deg), 0.0)
    dis_ref[...] = dis
    y_ref[...] = jnp.dot(x_ref[...], w_ref[...],
                         preferred_element_type=_f32) * dis


def _tc1_call(d0, d1, x_p, w1):
    return pl.pallas_call(
        _tc1_body,
        grid=(NPAD // _BT,),
        in_specs=[
            pl.BlockSpec((_BT, 1), lambda i: (i, 0)),
            pl.BlockSpec((_BT, 1), lambda i: (i, 0)),
            pl.BlockSpec((_BT, F_IN), lambda i: (i, 0)),
            pl.BlockSpec((F_IN, H), lambda i: (0, 0)),
        ],
        out_specs=[
            pl.BlockSpec((_BT, 1), lambda i: (i, 0)),
            pl.BlockSpec((_BT, H), lambda i: (i, 0)),
        ],
        out_shape=[
            jax.ShapeDtypeStruct((NPAD, 1), _f32),
            jax.ShapeDtypeStruct((NPAD, H), _f32),
        ],
    )(d0, d1, x_p, w1)


def _tc2_body(a0_ref, a1_ref, dis_ref, b_ref, w_ref, y_ref):
    dis = dis_ref[...]
    h = jax.nn.relu(dis * (a0_ref[...] + a1_ref[...]) + b_ref[...])
    y_ref[...] = jnp.dot(h, w_ref[...], preferred_element_type=_f32) * dis


def _tc2_call(a0, a1, dis, b, w):
    return pl.pallas_call(
        _tc2_body,
        grid=(NPAD // _BT,),
        in_specs=[
            pl.BlockSpec((_BT, H), lambda i: (i, 0)),
            pl.BlockSpec((_BT, H), lambda i: (i, 0)),
            pl.BlockSpec((_BT, 1), lambda i: (i, 0)),
            pl.BlockSpec((1, H), lambda i: (0, 0)),
            pl.BlockSpec((H, H), lambda i: (0, 0)),
        ],
        out_specs=pl.BlockSpec((_BT, H), lambda i: (i, 0)),
        out_shape=jax.ShapeDtypeStruct((NPAD, H), _f32),
    )(a0, a1, dis, b, w)


def _tcf_body(a0_ref, a1_ref, dis_ref, b3_ref, batch_ref, wl_ref, bl_ref,
              out_ref, sums, cnts):
    i = pl.program_id(0)

    @pl.when(i == 0)
    def _():
        sums[...] = jnp.zeros_like(sums)
        cnts[...] = jnp.zeros_like(cnts)

    h = dis_ref[...] * (a0_ref[...] + a1_ref[...]) + b3_ref[...]
    bt = batch_ref[...]
    onehot = (lax.broadcasted_iota(jnp.int32, (G, _BT), 0) == bt).astype(_f32)
    sums[...] += jnp.dot(onehot, h, preferred_element_type=_f32)
    cnts[...] += jnp.sum(onehot, axis=1, keepdims=True)

    @pl.when(i == pl.num_programs(0) - 1)
    def _():
        pooled = sums[...] / jnp.maximum(cnts[...], 1.0)
        out_ref[...] = jnp.dot(pooled, wl_ref[...],
                               preferred_element_type=_f32) + bl_ref[...]


def _tcf_call(a0, a1, dis, b3, batch_p, wl, bl):
    return pl.pallas_call(
        _tcf_body,
        grid=(NPAD // _BT,),
        in_specs=[
            pl.BlockSpec((_BT, H), lambda i: (i, 0)),
            pl.BlockSpec((_BT, H), lambda i: (i, 0)),
            pl.BlockSpec((_BT, 1), lambda i: (i, 0)),
            pl.BlockSpec((1, H), lambda i: (0, 0)),
            pl.BlockSpec((1, _BT), lambda i: (0, i)),
            pl.BlockSpec((H, OUT), lambda i: (0, 0)),
            pl.BlockSpec((1, OUT), lambda i: (0, 0)),
        ],
        out_specs=pl.BlockSpec((G, OUT), lambda i: (0, 0)),
        out_shape=jax.ShapeDtypeStruct((G, OUT), _f32),
        scratch_shapes=[
            pltpu.VMEM((G, H), _f32),
            pltpu.VMEM((G, 1), _f32),
        ],
    )(a0, a1, dis, b3, batch_p, wl, bl)


def kernel(x, edge_index, batch, W1, b1, W2, b2, W3, b3, Wl, bl):
    pad_e = PE - E
    src_p = jnp.concatenate(
        [edge_index[0], jnp.zeros((pad_e,), jnp.int32)]).reshape(NW, EC, CH)
    dst_p = jnp.concatenate(
        [edge_index[1], jnp.full((pad_e,), N, jnp.int32)]).reshape(NW, EC, CH)
    x_p = jnp.pad(x, ((0, NPAD - N), (0, 0)))
    batch_p = jnp.concatenate(
        [batch, jnp.full((NPAD - N,), G, jnp.int32)]).reshape(1, NPAD)

    degs = _deg_kernel(dst_p)
    d0 = degs[0].reshape(NPAD, 1)
    d1 = degs[1].reshape(NPAD, 1)
    dis, y1 = _tc1_call(d0, d1, x_p, W1)

    a = _mp_kernel(y1, src_p, dst_p)
    y2 = _tc2_call(a[0], a[1], dis, b1.reshape(1, H), W2)
    a = _mp_kernel(y2, src_p, dst_p)
    y3 = _tc2_call(a[0], a[1], dis, b2.reshape(1, H), W3)
    a = _mp_kernel(y3, src_p, dst_p)
    return _tcf_call(a[0], a[1], dis, b3.reshape(1, H), batch_p,
                     Wl, bl.reshape(1, OUT))


# speedup vs baseline: 1.7373x; 1.7373x over previous
import functools

import jax
import jax.numpy as jnp
from jax import lax
from jax.experimental import pallas as pl
from jax.experimental.pallas import tpu as pltpu
from jax.experimental.pallas import tpu_sc as plsc

N = 10000
E = 320000
F_IN = 128
H = 64
OUT = 128
G = 16

NC, NS, L = 2, 16, 16
NW = NC * NS
PT = 640
NPAD = NS * PT
CH = 128
EC = 80
PE = NW * EC * CH
NBUF = 4

_f32 = jnp.float32
_mesh = plsc.VectorSubcoreMesh(core_axis_name="c", subcore_axis_name="s")
_sc_params = pltpu.CompilerParams(use_tc_tiling_on_sc=False)


@functools.partial(
    pl.kernel,
    out_type=jax.ShapeDtypeStruct((NC, NPAD), _f32),
    mesh=_mesh,
    compiler_params=_sc_params,
    scratch_types=[
        pltpu.VMEM((EC, CH), jnp.int32),
        pltpu.VMEM((CH,), _f32),
        pltpu.VMEM((PT,), _f32),
        pltpu.VMEM_SHARED((NPAD,), _f32),
    ],
)
def _deg_kernel(dst_hbm, out_hbm, idx_d, ones_v, zbuf, acc):
    c = lax.axis_index("c")
    s = lax.axis_index("s")
    wid = c * NS + s
    pltpu.sync_copy(dst_hbm.at[wid], idx_d)
    for i in range(CH // L):
        ones_v[pl.ds(i * L, L)] = jnp.ones((L,), _f32)

    @pl.loop(0, PT // L)
    def _(i):
        zbuf[pl.ds(i * L, L)] = jnp.zeros((L,), _f32)

    pltpu.sync_copy(zbuf, acc.at[pl.ds(s * PT, PT)])
    plsc.subcore_barrier()

    @pl.loop(0, EC)
    def _(j):
        pltpu.sync_copy(ones_v, acc.at[idx_d.at[j]], add=True)

    plsc.subcore_barrier()
    pltpu.sync_copy(acc.at[pl.ds(s * PT, PT)], out_hbm.at[c, pl.ds(s * PT, PT)])


@functools.partial(
    pl.kernel,
    out_type=jax.ShapeDtypeStruct((NC, NPAD, H), _f32),
    mesh=_mesh,
    compiler_params=_sc_params,
    scratch_types=(
        [pltpu.VMEM((EC, CH), jnp.int32),
         pltpu.VMEM((EC, CH), jnp.int32)]
        + [pltpu.VMEM((CH, H), _f32) for _ in range(NBUF)]
        + [pltpu.VMEM_SHARED((NPAD, H), _f32)]
        + [pltpu.SemaphoreType.DMA for _ in range(2 * NBUF)]
    ),
)
def _mp_kernel(y_hbm, src_hbm, dst_hbm, out_hbm, idx_s, idx_d, *rest):
    bufs = rest[:NBUF]
    acc = rest[NBUF]
    gsems = rest[NBUF + 1:2 * NBUF + 1]
    ssems = rest[2 * NBUF + 1:]
    c = lax.axis_index("c")
    s = lax.axis_index("s")
    wid = c * NS + s
    pltpu.sync_copy(src_hbm.at[wid], idx_s)
    pltpu.sync_copy(dst_hbm.at[wid], idx_d)

    @pl.when(c == 0)
    def _():
        pltpu.sync_copy(y_hbm.at[pl.ds(s * PT, PT)], acc.at[pl.ds(s * PT, PT)])

    @pl.when(c == 1)
    def _():
        @pl.loop(0, CH)
        def _(i):
            for jj in range(H // L):
                bufs[0][i, pl.ds(jj * L, L)] = jnp.zeros((L,), _f32)

        @pl.loop(0, PT // CH)
        def _(k):
            pltpu.sync_copy(bufs[0], acc.at[pl.ds(s * PT + k * CH, CH)])

    plsc.subcore_barrier()

    def _gather(ch, b):
        return pltpu.make_async_copy(y_hbm.at[idx_s.at[ch]], bufs[b], gsems[b])

    def _scatter_start(ch, b):
        pltpu.async_copy(bufs[b], acc.at[idx_d.at[ch]], ssems[b], add=True)

    def _scatter_wait(b):
        pltpu.make_async_copy(bufs[b], acc.at[idx_d.at[0]], ssems[b]).wait()

    @pl.loop(0, EC)
    def _(j):
        pltpu.async_copy(y_hbm.at[idx_s.at[j]], bufs[0], gsems[0]).wait()
        _scatter_start(j, 0)
        _scatter_wait(0)

    plsc.subcore_barrier()
    pltpu.sync_copy(acc.at[pl.ds(s * PT, PT)], out_hbm.at[c, pl.ds(s * PT, PT)])


_BT = 2048


def _tc1_body(d0_ref, d1_ref, x_ref, w_ref, dis_ref, y_ref):
    i = pl.program_id(0)
    deg = d0_ref[...] + d1_ref[...] + 1.0
    rows = lax.broadcasted_iota(jnp.int32, (_BT, 1), 0) + i * _BT
    dis = jnp.where(rows < N, lax.rsqrt(deg), 0.0)
    dis_ref[...] = dis
    y_ref[...] = jnp.dot(x_ref[...], w_ref[...],
                         preferred_element_type=_f32) * dis


def _tc1_call(d0, d1, x_p, w1):
    return pl.pallas_call(
        _tc1_body,
        grid=(NPAD // _BT,),
        in_specs=[
            pl.BlockSpec((_BT, 1), lambda i: (i, 0)),
            pl.BlockSpec((_BT, 1), lambda i: (i, 0)),
            pl.BlockSpec((_BT, F_IN), lambda i: (i, 0)),
            pl.BlockSpec((F_IN, H), lambda i: (0, 0)),
        ],
        out_specs=[
            pl.BlockSpec((_BT, 1), lambda i: (i, 0)),
            pl.BlockSpec((_BT, H), lambda i: (i, 0)),
        ],
        out_shape=[
            jax.ShapeDtypeStruct((NPAD, 1), _f32),
            jax.ShapeDtypeStruct((NPAD, H), _f32),
        ],
    )(d0, d1, x_p, w1)


def _tc2_body(a0_ref, a1_ref, dis_ref, b_ref, w_ref, y_ref):
    dis = dis_ref[...]
    h = jax.nn.relu(dis * (a0_ref[...] + a1_ref[...]) + b_ref[...])
    y_ref[...] = jnp.dot(h, w_ref[...], preferred_element_type=_f32) * dis


def _tc2_call(a0, a1, dis, b, w):
    return pl.pallas_call(
        _tc2_body,
        grid=(NPAD // _BT,),
        in_specs=[
            pl.BlockSpec((_BT, H), lambda i: (i, 0)),
            pl.BlockSpec((_BT, H), lambda i: (i, 0)),
            pl.BlockSpec((_BT, 1), lambda i: (i, 0)),
            pl.BlockSpec((1, H), lambda i: (0, 0)),
            pl.BlockSpec((H, H), lambda i: (0, 0)),
        ],
        out_specs=pl.BlockSpec((_BT, H), lambda i: (i, 0)),
        out_shape=jax.ShapeDtypeStruct((NPAD, H), _f32),
    )(a0, a1, dis, b, w)


def _tcf_body(a0_ref, a1_ref, dis_ref, b3_ref, batch_ref, wl_ref, bl_ref,
              out_ref, sums, cnts):
    i = pl.program_id(0)

    @pl.when(i == 0)
    def _():
        sums[...] = jnp.zeros_like(sums)
        cnts[...] = jnp.zeros_like(cnts)

    h = dis_ref[...] * (a0_ref[...] + a1_ref[...]) + b3_ref[...]
    bt = batch_ref[...]
    onehot = (lax.broadcasted_iota(jnp.int32, (G, _BT), 0) == bt).astype(_f32)
    sums[...] += jnp.dot(onehot, h, preferred_element_type=_f32)
    cnts[...] += jnp.sum(onehot, axis=1, keepdims=True)

    @pl.when(i == pl.num_programs(0) - 1)
    def _():
        pooled = sums[...] / jnp.maximum(cnts[...], 1.0)
        out_ref[...] = jnp.dot(pooled, wl_ref[...],
                               preferred_element_type=_f32) + bl_ref[...]


def _tcf_call(a0, a1, dis, b3, batch_p, wl, bl):
    return pl.pallas_call(
        _tcf_body,
        grid=(NPAD // _BT,),
        in_specs=[
            pl.BlockSpec((_BT, H), lambda i: (i, 0)),
            pl.BlockSpec((_BT, H), lambda i: (i, 0)),
            pl.BlockSpec((_BT, 1), lambda i: (i, 0)),
            pl.BlockSpec((1, H), lambda i: (0, 0)),
            pl.BlockSpec((1, _BT), lambda i: (0, i)),
            pl.BlockSpec((H, OUT), lambda i: (0, 0)),
            pl.BlockSpec((1, OUT), lambda i: (0, 0)),
        ],
        out_specs=pl.BlockSpec((G, OUT), lambda i: (0, 0)),
        out_shape=jax.ShapeDtypeStruct((G, OUT), _f32),
        scratch_shapes=[
            pltpu.VMEM((G, H), _f32),
            pltpu.VMEM((G, 1), _f32),
        ],
    )(a0, a1, dis, b3, batch_p, wl, bl)


def kernel(x, edge_index, batch, W1, b1, W2, b2, W3, b3, Wl, bl):
    pad_e = PE - E
    pad_src = jnp.arange(pad_e, dtype=jnp.int32) % N
    pad_dst = N + (jnp.arange(pad_e, dtype=jnp.int32) % (NPAD - N))
    src_p = jnp.concatenate([edge_index[0], pad_src]) \
        .reshape(EC, NW, CH).transpose(1, 0, 2)
    dst_p = jnp.concatenate([edge_index[1], pad_dst]) \
        .reshape(EC, NW, CH).transpose(1, 0, 2)
    x_p = jnp.pad(x, ((0, NPAD - N), (0, 0)))
    batch_p = jnp.concatenate(
        [batch, jnp.full((NPAD - N,), G, jnp.int32)]).reshape(1, NPAD)

    degs = _deg_kernel(dst_p)
    d0 = degs[0].reshape(NPAD, 1)
    d1 = degs[1].reshape(NPAD, 1)
    dis, y1 = _tc1_call(d0, d1, x_p, W1)

    a = _mp_kernel(y1, src_p, dst_p)
    y2 = _tc2_call(a[0], a[1], dis, b1.reshape(1, H), W2)
    a = _mp_kernel(y2, src_p, dst_p)
    y3 = _tc2_call(a[0], a[1], dis, b2.reshape(1, H), W3)
    a = _mp_kernel(y3, src_p, dst_p)
    return _tcf_call(a[0], a[1], dis, b3.reshape(1, H), batch_p,
                     Wl, bl.reshape(1, OUT))


# speedup vs baseline: 2.6158x; 1.5057x over previous
import functools

import jax
import jax.numpy as jnp
from jax import lax
from jax.experimental import pallas as pl
from jax.experimental.pallas import tpu as pltpu
from jax.experimental.pallas import tpu_sc as plsc

N = 10000
E = 320000
F_IN = 128
H = 64
OUT = 128
G = 16

NC, NS, L = 2, 16, 16
NW = NC * NS
PT = 640
NPAD = NS * PT
CH = 128
EC = 80
PE = NW * EC * CH
NBUF = 4

_f32 = jnp.float32
_mesh = plsc.VectorSubcoreMesh(core_axis_name="c", subcore_axis_name="s")
_sc_params = pltpu.CompilerParams(use_tc_tiling_on_sc=False)


@functools.partial(
    pl.kernel,
    out_type=jax.ShapeDtypeStruct((NC, NPAD), _f32),
    mesh=_mesh,
    compiler_params=_sc_params,
    scratch_types=[
        pltpu.VMEM((EC, CH), jnp.int32),
        pltpu.VMEM((CH,), _f32),
        pltpu.VMEM((PT,), _f32),
        pltpu.VMEM_SHARED((NPAD,), _f32),
    ],
)
def _deg_kernel(dst_hbm, out_hbm, idx_d, ones_v, zbuf, acc):
    c = lax.axis_index("c")
    s = lax.axis_index("s")
    wid = c * NS + s
    pltpu.sync_copy(dst_hbm.at[wid], idx_d)
    for i in range(CH // L):
        ones_v[pl.ds(i * L, L)] = jnp.ones((L,), _f32)

    @pl.loop(0, PT // L)
    def _(i):
        zbuf[pl.ds(i * L, L)] = jnp.zeros((L,), _f32)

    pltpu.sync_copy(zbuf, acc.at[pl.ds(s * PT, PT)])
    plsc.subcore_barrier()

    @pl.loop(0, EC)
    def _(j):
        pltpu.sync_copy(ones_v, acc.at[idx_d.at[j]], add=True)

    plsc.subcore_barrier()
    pltpu.sync_copy(acc.at[pl.ds(s * PT, PT)], out_hbm.at[c, pl.ds(s * PT, PT)])


@functools.partial(
    pl.kernel,
    out_type=jax.ShapeDtypeStruct((NC, NPAD, H), _f32),
    mesh=_mesh,
    compiler_params=_sc_params,
    scratch_types=(
        [pltpu.VMEM((EC, CH), jnp.int32),
         pltpu.VMEM((EC, CH), jnp.int32)]
        + [pltpu.VMEM((CH, H), _f32) for _ in range(NBUF)]
        + [pltpu.VMEM_SHARED((NPAD, H), _f32)]
        + [pltpu.SemaphoreType.DMA for _ in range(2 * NBUF)]
    ),
)
def _mp_kernel(y_hbm, src_hbm, dst_hbm, out_hbm, idx_s, idx_d, *rest):
    bufs = rest[:NBUF]
    acc = rest[NBUF]
    gsems = rest[NBUF + 1:2 * NBUF + 1]
    ssems = rest[2 * NBUF + 1:]
    c = lax.axis_index("c")
    s = lax.axis_index("s")
    wid = c * NS + s
    pltpu.sync_copy(src_hbm.at[wid], idx_s)
    pltpu.sync_copy(dst_hbm.at[wid], idx_d)

    @pl.when(c == 0)
    def _():
        pltpu.sync_copy(y_hbm.at[pl.ds(s * PT, PT)], acc.at[pl.ds(s * PT, PT)])

    @pl.when(c == 1)
    def _():
        @pl.loop(0, CH)
        def _(i):
            for jj in range(H // L):
                bufs[0][i, pl.ds(jj * L, L)] = jnp.zeros((L,), _f32)

        @pl.loop(0, PT // CH)
        def _(k):
            pltpu.sync_copy(bufs[0], acc.at[pl.ds(s * PT + k * CH, CH)])

    plsc.subcore_barrier()

    def _gather(ch, b):
        return pltpu.make_async_copy(y_hbm.at[idx_s.at[ch]], bufs[b], gsems[b])

    def _scatter_start(ch, b):
        pltpu.async_copy(bufs[b], acc.at[idx_d.at[ch]], ssems[b], add=True)

    def _scatter_wait(b):
        pltpu.make_async_copy(bufs[b], acc.at[idx_d.at[0]], ssems[b]).wait()

    for b in range(NBUF):
        _gather(b, b).start()

    @pl.loop(0, EC - NBUF, step=NBUF)
    def _(j):
        for b in range(NBUF):
            _gather(j + b, b).wait()
            _scatter_start(j + b, b)
        for b in range(NBUF):
            _scatter_wait(b)
            _gather(j + NBUF + b, b).start()

    for b in range(NBUF):
        _gather(EC - NBUF + b, b).wait()
        _scatter_start(EC - NBUF + b, b)
    for b in range(NBUF):
        _scatter_wait(b)

    plsc.subcore_barrier()
    pltpu.sync_copy(acc.at[pl.ds(s * PT, PT)], out_hbm.at[c, pl.ds(s * PT, PT)])


_BT = 2048


def _tc1_body(d0_ref, d1_ref, x_ref, w_ref, dis_ref, y_ref):
    i = pl.program_id(0)
    deg = d0_ref[...] + d1_ref[...] + 1.0
    rows = lax.broadcasted_iota(jnp.int32, (_BT, 1), 0) + i * _BT
    dis = jnp.where(rows < N, lax.rsqrt(deg), 0.0)
    dis_ref[...] = dis
    y_ref[...] = jnp.dot(x_ref[...], w_ref[...],
                         preferred_element_type=_f32) * dis


def _tc1_call(d0, d1, x_p, w1):
    return pl.pallas_call(
        _tc1_body,
        grid=(NPAD // _BT,),
        in_specs=[
            pl.BlockSpec((_BT, 1), lambda i: (i, 0)),
            pl.BlockSpec((_BT, 1), lambda i: (i, 0)),
            pl.BlockSpec((_BT, F_IN), lambda i: (i, 0)),
            pl.BlockSpec((F_IN, H), lambda i: (0, 0)),
        ],
        out_specs=[
            pl.BlockSpec((_BT, 1), lambda i: (i, 0)),
            pl.BlockSpec((_BT, H), lambda i: (i, 0)),
        ],
        out_shape=[
            jax.ShapeDtypeStruct((NPAD, 1), _f32),
            jax.ShapeDtypeStruct((NPAD, H), _f32),
        ],
    )(d0, d1, x_p, w1)


def _tc2_body(a0_ref, a1_ref, dis_ref, b_ref, w_ref, y_ref):
    dis = dis_ref[...]
    h = jax.nn.relu(dis * (a0_ref[...] + a1_ref[...]) + b_ref[...])
    y_ref[...] = jnp.dot(h, w_ref[...], preferred_element_type=_f32) * dis


def _tc2_call(a0, a1, dis, b, w):
    return pl.pallas_call(
        _tc2_body,
        grid=(NPAD // _BT,),
        in_specs=[
            pl.BlockSpec((_BT, H), lambda i: (i, 0)),
            pl.BlockSpec((_BT, H), lambda i: (i, 0)),
            pl.BlockSpec((_BT, 1), lambda i: (i, 0)),
            pl.BlockSpec((1, H), lambda i: (0, 0)),
            pl.BlockSpec((H, H), lambda i: (0, 0)),
        ],
        out_specs=pl.BlockSpec((_BT, H), lambda i: (i, 0)),
        out_shape=jax.ShapeDtypeStruct((NPAD, H), _f32),
    )(a0, a1, dis, b, w)


def _tcf_body(a0_ref, a1_ref, dis_ref, b3_ref, batch_ref, wl_ref, bl_ref,
              out_ref, sums, cnts):
    i = pl.program_id(0)

    @pl.when(i == 0)
    def _():
        sums[...] = jnp.zeros_like(sums)
        cnts[...] = jnp.zeros_like(cnts)

    h = dis_ref[...] * (a0_ref[...] + a1_ref[...]) + b3_ref[...]
    bt = batch_ref[...]
    onehot = (lax.broadcasted_iota(jnp.int32, (G, _BT), 0) == bt).astype(_f32)
    sums[...] += jnp.dot(onehot, h, preferred_element_type=_f32)
    cnts[...] += jnp.sum(onehot, axis=1, keepdims=True)

    @pl.when(i == pl.num_programs(0) - 1)
    def _():
        pooled = sums[...] / jnp.maximum(cnts[...], 1.0)
        out_ref[...] = jnp.dot(pooled, wl_ref[...],
                               preferred_element_type=_f32) + bl_ref[...]


def _tcf_call(a0, a1, dis, b3, batch_p, wl, bl):
    return pl.pallas_call(
        _tcf_body,
        grid=(NPAD // _BT,),
        in_specs=[
            pl.BlockSpec((_BT, H), lambda i: (i, 0)),
            pl.BlockSpec((_BT, H), lambda i: (i, 0)),
            pl.BlockSpec((_BT, 1), lambda i: (i, 0)),
            pl.BlockSpec((1, H), lambda i: (0, 0)),
            pl.BlockSpec((1, _BT), lambda i: (0, i)),
            pl.BlockSpec((H, OUT), lambda i: (0, 0)),
            pl.BlockSpec((1, OUT), lambda i: (0, 0)),
        ],
        out_specs=pl.BlockSpec((G, OUT), lambda i: (0, 0)),
        out_shape=jax.ShapeDtypeStruct((G, OUT), _f32),
        scratch_shapes=[
            pltpu.VMEM((G, H), _f32),
            pltpu.VMEM((G, 1), _f32),
        ],
    )(a0, a1, dis, b3, batch_p, wl, bl)


def kernel(x, edge_index, batch, W1, b1, W2, b2, W3, b3, Wl, bl):
    pad_e = PE - E
    pad_src = jnp.arange(pad_e, dtype=jnp.int32) % N
    pad_dst = N + (jnp.arange(pad_e, dtype=jnp.int32) % (NPAD - N))
    src_p = jnp.concatenate([edge_index[0], pad_src]) \
        .reshape(EC, NW, CH).transpose(1, 0, 2)
    dst_p = jnp.concatenate([edge_index[1], pad_dst]) \
        .reshape(EC, NW, CH).transpose(1, 0, 2)
    x_p = jnp.pad(x, ((0, NPAD - N), (0, 0)))
    batch_p = jnp.concatenate(
        [batch, jnp.full((NPAD - N,), G, jnp.int32)]).reshape(1, NPAD)

    degs = _deg_kernel(dst_p)
    d0 = degs[0].reshape(NPAD, 1)
    d1 = degs[1].reshape(NPAD, 1)
    dis, y1 = _tc1_call(d0, d1, x_p, W1)

    a = _mp_kernel(y1, src_p, dst_p)
    y2 = _tc2_call(a[0], a[1], dis, b1.reshape(1, H), W2)
    a = _mp_kernel(y2, src_p, dst_p)
    y3 = _tc2_call(a[0], a[1], dis, b2.reshape(1, H), W3)
    a = _mp_kernel(y3, src_p, dst_p)
    return _tcf_call(a[0], a[1], dis, b3.reshape(1, H), batch_p,
                     Wl, bl.reshape(1, OUT))


# speedup vs baseline: 2.7019x; 1.0329x over previous
import functools

import jax
import jax.numpy as jnp
from jax import lax
from jax.experimental import pallas as pl
from jax.experimental.pallas import tpu as pltpu
from jax.experimental.pallas import tpu_sc as plsc

N = 10000
E = 320000
F_IN = 128
H = 64
OUT = 128
G = 16

NC, NS, L = 2, 16, 16
NW = NC * NS
PT = 640
NPAD = NS * PT
CH = 128
EC = 80
PE = NW * EC * CH
NBUF = 8

_f32 = jnp.float32
_mesh = plsc.VectorSubcoreMesh(core_axis_name="c", subcore_axis_name="s")
_sc_params = pltpu.CompilerParams(use_tc_tiling_on_sc=False)


@functools.partial(
    pl.kernel,
    out_type=jax.ShapeDtypeStruct((NC, NPAD), _f32),
    mesh=_mesh,
    compiler_params=_sc_params,
    scratch_types=[
        pltpu.VMEM((EC, CH), jnp.int32),
        pltpu.VMEM((CH,), _f32),
        pltpu.VMEM((PT,), _f32),
        pltpu.VMEM_SHARED((NPAD,), _f32),
    ],
)
def _deg_kernel(dst_hbm, out_hbm, idx_d, ones_v, zbuf, acc):
    c = lax.axis_index("c")
    s = lax.axis_index("s")
    wid = c * NS + s
    pltpu.sync_copy(dst_hbm.at[wid], idx_d)
    for i in range(CH // L):
        ones_v[pl.ds(i * L, L)] = jnp.ones((L,), _f32)

    @pl.loop(0, PT // L)
    def _(i):
        zbuf[pl.ds(i * L, L)] = jnp.zeros((L,), _f32)

    pltpu.sync_copy(zbuf, acc.at[pl.ds(s * PT, PT)])
    plsc.subcore_barrier()

    @pl.loop(0, EC)
    def _(j):
        pltpu.sync_copy(ones_v, acc.at[idx_d.at[j]], add=True)

    plsc.subcore_barrier()
    pltpu.sync_copy(acc.at[pl.ds(s * PT, PT)], out_hbm.at[c, pl.ds(s * PT, PT)])


@functools.partial(
    pl.kernel,
    out_type=jax.ShapeDtypeStruct((NC, NPAD, H), _f32),
    mesh=_mesh,
    compiler_params=_sc_params,
    scratch_types=(
        [pltpu.VMEM((EC, CH), jnp.int32),
         pltpu.VMEM((EC, CH), jnp.int32)]
        + [pltpu.VMEM((CH, H), _f32) for _ in range(NBUF)]
        + [pltpu.VMEM_SHARED((NPAD, H), _f32)]
        + [pltpu.SemaphoreType.DMA for _ in range(2 * NBUF)]
    ),
)
def _mp_kernel(y_hbm, src_hbm, dst_hbm, out_hbm, idx_s, idx_d, *rest):
    bufs = rest[:NBUF]
    acc = rest[NBUF]
    gsems = rest[NBUF + 1:2 * NBUF + 1]
    ssems = rest[2 * NBUF + 1:]
    c = lax.axis_index("c")
    s = lax.axis_index("s")
    wid = c * NS + s
    pltpu.sync_copy(src_hbm.at[wid], idx_s)
    pltpu.sync_copy(dst_hbm.at[wid], idx_d)

    @pl.when(c == 0)
    def _():
        pltpu.sync_copy(y_hbm.at[pl.ds(s * PT, PT)], acc.at[pl.ds(s * PT, PT)])

    @pl.when(c == 1)
    def _():
        @pl.loop(0, CH)
        def _(i):
            for jj in range(H // L):
                bufs[0][i, pl.ds(jj * L, L)] = jnp.zeros((L,), _f32)

        @pl.loop(0, PT // CH)
        def _(k):
            pltpu.sync_copy(bufs[0], acc.at[pl.ds(s * PT + k * CH, CH)])

    plsc.subcore_barrier()

    def _gather(ch, b):
        return pltpu.make_async_copy(y_hbm.at[idx_s.at[ch]], bufs[b], gsems[b])

    def _scatter_start(ch, b):
        pltpu.async_copy(bufs[b], acc.at[idx_d.at[ch]], ssems[b], add=True)

    def _scatter_wait(b):
        pltpu.make_async_copy(bufs[b], acc.at[idx_d.at[0]], ssems[b]).wait()

    for b in range(NBUF):
        _gather(b, b).start()

    @pl.loop(0, EC - NBUF, step=NBUF)
    def _(j):
        for b in range(NBUF):
            _gather(j + b, b).wait()
            _scatter_start(j + b, b)
        for b in range(NBUF):
            _scatter_wait(b)
            _gather(j + NBUF + b, b).start()

    for b in range(NBUF):
        _gather(EC - NBUF + b, b).wait()
        _scatter_start(EC - NBUF + b, b)
    for b in range(NBUF):
        _scatter_wait(b)

    plsc.subcore_barrier()
    pltpu.sync_copy(acc.at[pl.ds(s * PT, PT)], out_hbm.at[c, pl.ds(s * PT, PT)])


_BT = 2048


def _tc1_body(d0_ref, d1_ref, x_ref, w_ref, dis_ref, y_ref):
    i = pl.program_id(0)
    deg = d0_ref[...] + d1_ref[...] + 1.0
    rows = lax.broadcasted_iota(jnp.int32, (_BT, 1), 0) + i * _BT
    dis = jnp.where(rows < N, lax.rsqrt(deg), 0.0)
    dis_ref[...] = dis
    y_ref[...] = jnp.dot(x_ref[...], w_ref[...],
                         preferred_element_type=_f32) * dis


def _tc1_call(d0, d1, x_p, w1):
    return pl.pallas_call(
        _tc1_body,
        grid=(NPAD // _BT,),
        in_specs=[
            pl.BlockSpec((_BT, 1), lambda i: (i, 0)),
            pl.BlockSpec((_BT, 1), lambda i: (i, 0)),
            pl.BlockSpec((_BT, F_IN), lambda i: (i, 0)),
            pl.BlockSpec((F_IN, H), lambda i: (0, 0)),
        ],
        out_specs=[
            pl.BlockSpec((_BT, 1), lambda i: (i, 0)),
            pl.BlockSpec((_BT, H), lambda i: (i, 0)),
        ],
        out_shape=[
            jax.ShapeDtypeStruct((NPAD, 1), _f32),
            jax.ShapeDtypeStruct((NPAD, H), _f32),
        ],
    )(d0, d1, x_p, w1)


def _tc2_body(a0_ref, a1_ref, dis_ref, b_ref, w_ref, y_ref):
    dis = dis_ref[...]
    h = jax.nn.relu(dis * (a0_ref[...] + a1_ref[...]) + b_ref[...])
    y_ref[...] = jnp.dot(h, w_ref[...], preferred_element_type=_f32) * dis


def _tc2_call(a0, a1, dis, b, w):
    return pl.pallas_call(
        _tc2_body,
        grid=(NPAD // _BT,),
        in_specs=[
            pl.BlockSpec((_BT, H), lambda i: (i, 0)),
            pl.BlockSpec((_BT, H), lambda i: (i, 0)),
            pl.BlockSpec((_BT, 1), lambda i: (i, 0)),
            pl.BlockSpec((1, H), lambda i: (0, 0)),
            pl.BlockSpec((H, H), lambda i: (0, 0)),
        ],
        out_specs=pl.BlockSpec((_BT, H), lambda i: (i, 0)),
        out_shape=jax.ShapeDtypeStruct((NPAD, H), _f32),
    )(a0, a1, dis, b, w)


def _tcf_body(a0_ref, a1_ref, dis_ref, b3_ref, batch_ref, wl_ref, bl_ref,
              out_ref, sums, cnts):
    i = pl.program_id(0)

    @pl.when(i == 0)
    def _():
        sums[...] = jnp.zeros_like(sums)
        cnts[...] = jnp.zeros_like(cnts)

    h = dis_ref[...] * (a0_ref[...] + a1_ref[...]) + b3_ref[...]
    bt = batch_ref[...]
    onehot = (lax.broadcasted_iota(jnp.int32, (G, _BT), 0) == bt).astype(_f32)
    sums[...] += jnp.dot(onehot, h, preferred_element_type=_f32)
    cnts[...] += jnp.sum(onehot, axis=1, keepdims=True)

    @pl.when(i == pl.num_programs(0) - 1)
    def _():
        pooled = sums[...] / jnp.maximum(cnts[...], 1.0)
        out_ref[...] = jnp.dot(pooled, wl_ref[...],
                               preferred_element_type=_f32) + bl_ref[...]


def _tcf_call(a0, a1, dis, b3, batch_p, wl, bl):
    return pl.pallas_call(
        _tcf_body,
        grid=(NPAD // _BT,),
        in_specs=[
            pl.BlockSpec((_BT, H), lambda i: (i, 0)),
            pl.BlockSpec((_BT, H), lambda i: (i, 0)),
            pl.BlockSpec((_BT, 1), lambda i: (i, 0)),
            pl.BlockSpec((1, H), lambda i: (0, 0)),
            pl.BlockSpec((1, _BT), lambda i: (0, i)),
            pl.BlockSpec((H, OUT), lambda i: (0, 0)),
            pl.BlockSpec((1, OUT), lambda i: (0, 0)),
        ],
        out_specs=pl.BlockSpec((G, OUT), lambda i: (0, 0)),
        out_shape=jax.ShapeDtypeStruct((G, OUT), _f32),
        scratch_shapes=[
            pltpu.VMEM((G, H), _f32),
            pltpu.VMEM((G, 1), _f32),
        ],
    )(a0, a1, dis, b3, batch_p, wl, bl)


def kernel(x, edge_index, batch, W1, b1, W2, b2, W3, b3, Wl, bl):
    pad_e = PE - E
    pad_src = jnp.arange(pad_e, dtype=jnp.int32) % N
    pad_dst = N + (jnp.arange(pad_e, dtype=jnp.int32) % (NPAD - N))
    src_p = jnp.concatenate([edge_index[0], pad_src]) \
        .reshape(EC, NW, CH).transpose(1, 0, 2)
    dst_p = jnp.concatenate([edge_index[1], pad_dst]) \
        .reshape(EC, NW, CH).transpose(1, 0, 2)
    x_p = jnp.pad(x, ((0, NPAD - N), (0, 0)))
    batch_p = jnp.concatenate(
        [batch, jnp.full((NPAD - N,), G, jnp.int32)]).reshape(1, NPAD)

    degs = _deg_kernel(dst_p)
    d0 = degs[0].reshape(NPAD, 1)
    d1 = degs[1].reshape(NPAD, 1)
    dis, y1 = _tc1_call(d0, d1, x_p, W1)

    a = _mp_kernel(y1, src_p, dst_p)
    y2 = _tc2_call(a[0], a[1], dis, b1.reshape(1, H), W2)
    a = _mp_kernel(y2, src_p, dst_p)
    y3 = _tc2_call(a[0], a[1], dis, b2.reshape(1, H), W3)
    a = _mp_kernel(y3, src_p, dst_p)
    return _tcf_call(a[0], a[1], dis, b3.reshape(1, H), batch_p,
                     Wl, bl.reshape(1, OUT))


# speedup vs baseline: 3.2609x; 1.2069x over previous
import functools

import jax
import jax.numpy as jnp
from jax import lax
from jax.experimental import pallas as pl
from jax.experimental.pallas import tpu as pltpu
from jax.experimental.pallas import tpu_sc as plsc

N = 10000
E = 320000
F_IN = 128
H = 64
OUT = 128
G = 16

NC, NS, L = 2, 16, 16
NW = NC * NS
PT = 640
NPAD = NS * PT
CH = 128
EC = 80
PE = NW * EC * CH
NBUF = 8

_f32 = jnp.float32
_mesh = plsc.VectorSubcoreMesh(core_axis_name="c", subcore_axis_name="s")
_sc_params = pltpu.CompilerParams(use_tc_tiling_on_sc=False)


@functools.partial(
    pl.kernel,
    out_type=jax.ShapeDtypeStruct((NC, NPAD), _f32),
    mesh=_mesh,
    compiler_params=_sc_params,
    scratch_types=[
        pltpu.VMEM((EC, CH), jnp.int32),
        pltpu.VMEM((CH,), _f32),
        pltpu.VMEM((PT,), _f32),
        pltpu.VMEM_SHARED((NPAD,), _f32),
    ],
)
def _deg_kernel(dst_hbm, out_hbm, idx_d, ones_v, zbuf, acc):
    c = lax.axis_index("c")
    s = lax.axis_index("s")
    wid = c * NS + s
    pltpu.sync_copy(dst_hbm.at[wid], idx_d)
    for i in range(CH // L):
        ones_v[pl.ds(i * L, L)] = jnp.ones((L,), _f32)

    @pl.loop(0, PT // L)
    def _(i):
        zbuf[pl.ds(i * L, L)] = jnp.zeros((L,), _f32)

    pltpu.sync_copy(zbuf, acc.at[pl.ds(s * PT, PT)])
    plsc.subcore_barrier()

    @pl.loop(0, EC)
    def _(j):
        pltpu.sync_copy(ones_v, acc.at[idx_d.at[j]], add=True)

    plsc.subcore_barrier()
    pltpu.sync_copy(acc.at[pl.ds(s * PT, PT)], out_hbm.at[c, pl.ds(s * PT, PT)])


@functools.partial(
    pl.kernel,
    out_type=jax.ShapeDtypeStruct((NC, NPAD, H), _f32),
    mesh=_mesh,
    compiler_params=_sc_params,
    scratch_types=(
        [pltpu.VMEM((EC, CH), jnp.int32),
         pltpu.VMEM((EC, CH), jnp.int32)]
        + [pltpu.VMEM((CH, H), _f32) for _ in range(NBUF)]
        + [pltpu.VMEM_SHARED((NPAD, H), _f32)]
        + [pltpu.SemaphoreType.DMA for _ in range(2 * NBUF)]
    ),
)
def _mp_kernel(y_hbm, src_hbm, dst_hbm, out_hbm, idx_s, idx_d, *rest):
    bufs = rest[:NBUF]
    acc = rest[NBUF]
    gsems = rest[NBUF + 1:2 * NBUF + 1]
    ssems = rest[2 * NBUF + 1:]
    c = lax.axis_index("c")
    s = lax.axis_index("s")
    wid = c * NS + s
    pltpu.sync_copy(src_hbm.at[wid], idx_s)
    pltpu.sync_copy(dst_hbm.at[wid], idx_d)

    @pl.when(c == 0)
    def _():
        pltpu.sync_copy(y_hbm.at[pl.ds(s * PT, PT)], acc.at[pl.ds(s * PT, PT)])

    @pl.when(c == 1)
    def _():
        @pl.loop(0, CH)
        def _(i):
            for jj in range(H // L):
                bufs[0][i, pl.ds(jj * L, L)] = jnp.zeros((L,), _f32)

        @pl.loop(0, PT // CH)
        def _(k):
            pltpu.sync_copy(bufs[0], acc.at[pl.ds(s * PT + k * CH, CH)])

    plsc.subcore_barrier()

    def _gather(ch, b):
        return pltpu.make_async_copy(y_hbm.at[idx_s.at[ch]], bufs[b], gsems[b])

    def _scatter_start(ch, b):
        pltpu.async_copy(bufs[b], acc.at[idx_d.at[ch]], ssems[b], add=True)

    def _scatter_wait(b):
        pltpu.make_async_copy(bufs[b], acc.at[idx_d.at[0]], ssems[b]).wait()

    for b in range(NBUF):
        _gather(b, b).start()

    @pl.loop(0, EC - NBUF, step=NBUF)
    def _(j):
        for b in range(NBUF):
            _gather(j + b, b).wait()
            _scatter_start(j + b, b)
        for b in range(NBUF):
            _scatter_wait(b)
            _gather(j + NBUF + b, b).start()

    for b in range(NBUF):
        _gather(EC - NBUF + b, b).wait()
        _scatter_start(EC - NBUF + b, b)
    for b in range(NBUF):
        _scatter_wait(b)

    plsc.subcore_barrier()
    pltpu.sync_copy(acc.at[pl.ds(s * PT, PT)], out_hbm.at[c, pl.ds(s * PT, PT)])


NP2 = NPAD // 2
_BT = 1024
LW = 2 * H


def _tc1_body(d0e, d1e, d0o, d1o, xe, xo, w1l, w1r, dis_ref, y_ref):
    i = pl.program_id(0)
    rows = lax.broadcasted_iota(jnp.int32, (_BT, 1), 0) + i * _BT
    dis_e = jnp.where(2 * rows < N,
                      lax.rsqrt(d0e[...] + d1e[...] + 1.0), 0.0)
    dis_o = jnp.where(2 * rows + 1 < N,
                      lax.rsqrt(d0o[...] + d1o[...] + 1.0), 0.0)
    dis = jnp.concatenate([jnp.broadcast_to(dis_e, (_BT, H)),
                           jnp.broadcast_to(dis_o, (_BT, H))], axis=1)
    dis_ref[...] = dis
    xw = (jnp.dot(xe[...], w1l[...], preferred_element_type=_f32)
          + jnp.dot(xo[...], w1r[...], preferred_element_type=_f32))
    y_ref[...] = xw * dis


def _tc1_call(d0e, d1e, d0o, d1o, xe, xo, w1l, w1r):
    col = pl.BlockSpec((_BT, 1), lambda i: (i, 0))
    full = pl.BlockSpec((_BT, LW), lambda i: (i, 0))
    wspec = pl.BlockSpec((LW, LW), lambda i: (0, 0))
    return pl.pallas_call(
        _tc1_body,
        grid=(NP2 // _BT,),
        in_specs=[col, col, col, col, full, full, wspec, wspec],
        out_specs=[full, full],
        out_shape=[
            jax.ShapeDtypeStruct((NP2, LW), _f32),
            jax.ShapeDtypeStruct((NP2, LW), _f32),
        ],
    )(d0e, d1e, d0o, d1o, xe, xo, w1l, w1r)


def _tc2_body(a0_ref, a1_ref, dis_ref, b_ref, w_ref, y_ref):
    dis = dis_ref[...]
    a0 = a0_ref[...].reshape(_BT, LW)
    a1 = a1_ref[...].reshape(_BT, LW)
    h = jax.nn.relu(dis * (a0 + a1) + b_ref[...])
    y_ref[...] = jnp.dot(h, w_ref[...], preferred_element_type=_f32) * dis


def _tc2_call(a128, dis, b128, wd):
    full = pl.BlockSpec((_BT, LW), lambda i: (i, 0))
    return pl.pallas_call(
        _tc2_body,
        grid=(NP2 // _BT,),
        in_specs=[
            pl.BlockSpec((1, _BT, LW), lambda i: (0, i, 0)),
            pl.BlockSpec((1, _BT, LW), lambda i: (1, i, 0)),
            full,
            pl.BlockSpec((1, LW), lambda i: (0, 0)),
            pl.BlockSpec((LW, LW), lambda i: (0, 0)),
        ],
        out_specs=full,
        out_shape=jax.ShapeDtypeStruct((NP2, LW), _f32),
    )(a128, a128, dis, b128, wd)


def _tcf_body(a0_ref, a1_ref, dis_ref, b3_ref, bte_ref, bto_ref,
              wl_ref, bl_ref, out_ref, sums, cnts):
    i = pl.program_id(0)

    @pl.when(i == 0)
    def _():
        sums[...] = jnp.zeros_like(sums)
        cnts[...] = jnp.zeros_like(cnts)

    a0 = a0_ref[...].reshape(_BT, LW)
    a1 = a1_ref[...].reshape(_BT, LW)
    h = dis_ref[...] * (a0 + a1) + b3_ref[...]
    giota = lax.broadcasted_iota(jnp.int32, (G, _BT), 0)
    oh_e = (giota == bte_ref[...]).astype(_f32)
    oh_o = (giota == bto_ref[...]).astype(_f32)
    sums[...] += (jnp.dot(oh_e, h[:, :H], preferred_element_type=_f32)
                  + jnp.dot(oh_o, h[:, H:], preferred_element_type=_f32))
    cnts[...] += (jnp.sum(oh_e, axis=1, keepdims=True)
                  + jnp.sum(oh_o, axis=1, keepdims=True))

    @pl.when(i == pl.num_programs(0) - 1)
    def _():
        pooled = sums[...] / jnp.maximum(cnts[...], 1.0)
        out_ref[...] = jnp.dot(pooled, wl_ref[...],
                               preferred_element_type=_f32) + bl_ref[...]


def _tcf_call(a128, dis, b3_128, bte, bto, wl, bl):
    full = pl.BlockSpec((_BT, LW), lambda i: (i, 0))
    return pl.pallas_call(
        _tcf_body,
        grid=(NP2 // _BT,),
        in_specs=[
            pl.BlockSpec((1, _BT, LW), lambda i: (0, i, 0)),
            pl.BlockSpec((1, _BT, LW), lambda i: (1, i, 0)),
            full,
            pl.BlockSpec((1, LW), lambda i: (0, 0)),
            pl.BlockSpec((1, _BT), lambda i: (0, i)),
            pl.BlockSpec((1, _BT), lambda i: (0, i)),
            pl.BlockSpec((H, OUT), lambda i: (0, 0)),
            pl.BlockSpec((1, OUT), lambda i: (0, 0)),
        ],
        out_specs=pl.BlockSpec((G, OUT), lambda i: (0, 0)),
        out_shape=jax.ShapeDtypeStruct((G, OUT), _f32),
        scratch_shapes=[
            pltpu.VMEM((G, H), _f32),
            pltpu.VMEM((G, 1), _f32),
        ],
    )(a128, a128, dis, b3_128, bte, bto, wl, bl)


def _blockdiag(w):
    z = jnp.zeros((H, H), _f32)
    return jnp.concatenate(
        [jnp.concatenate([w, z], axis=1),
         jnp.concatenate([z, w], axis=1)], axis=0)


def kernel(x, edge_index, batch, W1, b1, W2, b2, W3, b3, Wl, bl):
    pad_e = PE - E
    pad_src = jnp.arange(pad_e, dtype=jnp.int32) % N
    pad_dst = N + (jnp.arange(pad_e, dtype=jnp.int32) % (NPAD - N))
    src_p = jnp.concatenate([edge_index[0], pad_src]) \
        .reshape(EC, NW, CH).transpose(1, 0, 2)
    dst_p = jnp.concatenate([edge_index[1], pad_dst]) \
        .reshape(EC, NW, CH).transpose(1, 0, 2)
    x_p = jnp.pad(x, ((0, NPAD - N), (0, 0)))
    xe = x_p[0::2]
    xo = x_p[1::2]
    batch_p = jnp.concatenate(
        [batch, jnp.full((NPAD - N,), G, jnp.int32)])
    bte = batch_p[0::2].reshape(1, NP2)
    bto = batch_p[1::2].reshape(1, NP2)

    degs = _deg_kernel(dst_p)
    d0e = degs[0, 0::2].reshape(NP2, 1)
    d0o = degs[0, 1::2].reshape(NP2, 1)
    d1e = degs[1, 0::2].reshape(NP2, 1)
    d1o = degs[1, 1::2].reshape(NP2, 1)
    zh = jnp.zeros((F_IN, H), _f32)
    w1l = jnp.concatenate([W1, zh], axis=1)
    w1r = jnp.concatenate([zh, W1], axis=1)
    dis, y1 = _tc1_call(d0e, d1e, d0o, d1o, xe, xo, w1l, w1r)

    w2d = _blockdiag(W2)
    w3d = _blockdiag(W3)
    b1_128 = jnp.tile(b1, 2).reshape(1, LW)
    b2_128 = jnp.tile(b2, 2).reshape(1, LW)
    b3_128 = jnp.tile(b3, 2).reshape(1, LW)

    a = _mp_kernel(y1.reshape(NPAD, H), src_p, dst_p)
    y2 = _tc2_call(a.reshape(NC, NP2, LW), dis, b1_128, w2d)
    a = _mp_kernel(y2.reshape(NPAD, H), src_p, dst_p)
    y3 = _tc2_call(a.reshape(NC, NP2, LW), dis, b2_128, w3d)
    a = _mp_kernel(y3.reshape(NPAD, H), src_p, dst_p)
    return _tcf_call(a.reshape(NC, NP2, LW), dis, b3_128, bte, bto,
                     Wl, bl.reshape(1, OUT))


# speedup vs baseline: 3.3080x; 1.0145x over previous
import functools

import jax
import jax.numpy as jnp
from jax import lax
from jax.experimental import pallas as pl
from jax.experimental.pallas import tpu as pltpu
from jax.experimental.pallas import tpu_sc as plsc

N = 10000
E = 320000
F_IN = 128
H = 64
OUT = 128
G = 16

NC, NS, L = 2, 16, 16
NW = NC * NS
PT = 640
NPAD = NS * PT
CH = 128
EC = 80
PE = NW * EC * CH
NBUF = 8

_f32 = jnp.float32
_mesh = plsc.VectorSubcoreMesh(core_axis_name="c", subcore_axis_name="s")
_sc_params = pltpu.CompilerParams(use_tc_tiling_on_sc=False)


@functools.partial(
    pl.kernel,
    out_type=jax.ShapeDtypeStruct((NC, NPAD), _f32),
    mesh=_mesh,
    compiler_params=_sc_params,
    scratch_types=[
        pltpu.VMEM((EC, CH), jnp.int32),
        pltpu.VMEM((CH,), _f32),
        pltpu.VMEM((PT,), _f32),
        pltpu.VMEM_SHARED((NPAD,), _f32),
    ],
)
def _deg_kernel(dst_hbm, out_hbm, idx_d, ones_v, zbuf, acc):
    c = lax.axis_index("c")
    s = lax.axis_index("s")
    wid = c * NS + s
    pltpu.sync_copy(dst_hbm.at[:, wid], idx_d)
    for i in range(CH // L):
        ones_v[pl.ds(i * L, L)] = jnp.ones((L,), _f32)

    @pl.loop(0, PT // L)
    def _(i):
        zbuf[pl.ds(i * L, L)] = jnp.zeros((L,), _f32)

    pltpu.sync_copy(zbuf, acc.at[pl.ds(s * PT, PT)])
    plsc.subcore_barrier()

    @pl.loop(0, EC)
    def _(j):
        pltpu.sync_copy(ones_v, acc.at[idx_d.at[j]], add=True)

    plsc.subcore_barrier()
    pltpu.sync_copy(acc.at[pl.ds(s * PT, PT)], out_hbm.at[c, pl.ds(s * PT, PT)])


@functools.partial(
    pl.kernel,
    out_type=jax.ShapeDtypeStruct((NC, NPAD, H), _f32),
    mesh=_mesh,
    compiler_params=_sc_params,
    scratch_types=(
        [pltpu.VMEM((EC, CH), jnp.int32),
         pltpu.VMEM((EC, CH), jnp.int32)]
        + [pltpu.VMEM((CH, H), _f32) for _ in range(NBUF)]
        + [pltpu.VMEM_SHARED((NPAD, H), _f32)]
        + [pltpu.SemaphoreType.DMA for _ in range(2 * NBUF)]
    ),
)
def _mp_kernel(y_hbm, src_hbm, dst_hbm, out_hbm, idx_s, idx_d, *rest):
    bufs = rest[:NBUF]
    acc = rest[NBUF]
    gsems = rest[NBUF + 1:2 * NBUF + 1]
    ssems = rest[2 * NBUF + 1:]
    c = lax.axis_index("c")
    s = lax.axis_index("s")
    wid = c * NS + s
    pltpu.sync_copy(src_hbm.at[:, wid], idx_s)
    pltpu.sync_copy(dst_hbm.at[:, wid], idx_d)

    @pl.when(c == 0)
    def _():
        pltpu.sync_copy(y_hbm.at[pl.ds(s * PT, PT)], acc.at[pl.ds(s * PT, PT)])

    @pl.when(c == 1)
    def _():
        @pl.loop(0, CH)
        def _(i):
            for jj in range(H // L):
                bufs[0][i, pl.ds(jj * L, L)] = jnp.zeros((L,), _f32)

        @pl.loop(0, PT // CH)
        def _(k):
            pltpu.sync_copy(bufs[0], acc.at[pl.ds(s * PT + k * CH, CH)])

    plsc.subcore_barrier()

    def _gather(ch, b):
        return pltpu.make_async_copy(y_hbm.at[idx_s.at[ch]], bufs[b], gsems[b])

    def _scatter_start(ch, b):
        pltpu.async_copy(bufs[b], acc.at[idx_d.at[ch]], ssems[b], add=True)

    def _scatter_wait(b):
        pltpu.make_async_copy(bufs[b], acc.at[idx_d.at[0]], ssems[b]).wait()

    for b in range(NBUF):
        _gather(b, b).start()

    @pl.loop(0, EC - NBUF, step=NBUF)
    def _(j):
        for b in range(NBUF):
            _gather(j + b, b).wait()
            _scatter_start(j + b, b)
        for b in range(NBUF):
            _scatter_wait(b)
            _gather(j + NBUF + b, b).start()

    for b in range(NBUF):
        _gather(EC - NBUF + b, b).wait()
        _scatter_start(EC - NBUF + b, b)
    for b in range(NBUF):
        _scatter_wait(b)

    plsc.subcore_barrier()
    pltpu.sync_copy(acc.at[pl.ds(s * PT, PT)], out_hbm.at[c, pl.ds(s * PT, PT)])


NP2 = NPAD // 2
_BT = 1024
LW = 2 * H


def _tc1_body(d0e, d1e, d0o, d1o, xe, xo, w1l, w1r, dis_ref, y_ref):
    i = pl.program_id(0)
    rows = lax.broadcasted_iota(jnp.int32, (_BT, 1), 0) + i * _BT
    dis_e = jnp.where(2 * rows < N,
                      lax.rsqrt(d0e[...] + d1e[...] + 1.0), 0.0)
    dis_o = jnp.where(2 * rows + 1 < N,
                      lax.rsqrt(d0o[...] + d1o[...] + 1.0), 0.0)
    dis = jnp.concatenate([jnp.broadcast_to(dis_e, (_BT, H)),
                           jnp.broadcast_to(dis_o, (_BT, H))], axis=1)
    dis_ref[...] = dis
    xw = (jnp.dot(xe[...], w1l[...], preferred_element_type=_f32)
          + jnp.dot(xo[...], w1r[...], preferred_element_type=_f32))
    y_ref[...] = xw * dis


def _tc1_call(d0e, d1e, d0o, d1o, xe, xo, w1l, w1r):
    col = pl.BlockSpec((_BT, 1), lambda i: (i, 0))
    full = pl.BlockSpec((_BT, LW), lambda i: (i, 0))
    wspec = pl.BlockSpec((LW, LW), lambda i: (0, 0))
    return pl.pallas_call(
        _tc1_body,
        grid=(NP2 // _BT,),
        in_specs=[col, col, col, col, full, full, wspec, wspec],
        out_specs=[full, full],
        out_shape=[
            jax.ShapeDtypeStruct((NP2, LW), _f32),
            jax.ShapeDtypeStruct((NP2, LW), _f32),
        ],
    )(d0e, d1e, d0o, d1o, xe, xo, w1l, w1r)


def _tc2_body(a0_ref, a1_ref, dis_ref, b_ref, w_ref, y_ref):
    dis = dis_ref[...]
    a0 = a0_ref[...].reshape(_BT, LW)
    a1 = a1_ref[...].reshape(_BT, LW)
    h = jax.nn.relu(dis * (a0 + a1) + b_ref[...])
    y_ref[...] = jnp.dot(h, w_ref[...], preferred_element_type=_f32) * dis


def _tc2_call(a128, dis, b128, wd):
    full = pl.BlockSpec((_BT, LW), lambda i: (i, 0))
    return pl.pallas_call(
        _tc2_body,
        grid=(NP2 // _BT,),
        in_specs=[
            pl.BlockSpec((1, _BT, LW), lambda i: (0, i, 0)),
            pl.BlockSpec((1, _BT, LW), lambda i: (1, i, 0)),
            full,
            pl.BlockSpec((1, LW), lambda i: (0, 0)),
            pl.BlockSpec((LW, LW), lambda i: (0, 0)),
        ],
        out_specs=full,
        out_shape=jax.ShapeDtypeStruct((NP2, LW), _f32),
    )(a128, a128, dis, b128, wd)


def _tcf_body(a0_ref, a1_ref, dis_ref, b3_ref, bte_ref, bto_ref,
              wl_ref, bl_ref, out_ref, sums, cnts):
    i = pl.program_id(0)

    @pl.when(i == 0)
    def _():
        sums[...] = jnp.zeros_like(sums)
        cnts[...] = jnp.zeros_like(cnts)

    a0 = a0_ref[...].reshape(_BT, LW)
    a1 = a1_ref[...].reshape(_BT, LW)
    h = dis_ref[...] * (a0 + a1) + b3_ref[...]
    giota = lax.broadcasted_iota(jnp.int32, (G, _BT), 0)
    oh_e = (giota == bte_ref[...]).astype(_f32)
    oh_o = (giota == bto_ref[...]).astype(_f32)
    sums[...] += (jnp.dot(oh_e, h[:, :H], preferred_element_type=_f32)
                  + jnp.dot(oh_o, h[:, H:], preferred_element_type=_f32))
    cnts[...] += (jnp.sum(oh_e, axis=1, keepdims=True)
                  + jnp.sum(oh_o, axis=1, keepdims=True))

    @pl.when(i == pl.num_programs(0) - 1)
    def _():
        pooled = sums[...] / jnp.maximum(cnts[...], 1.0)
        out_ref[...] = jnp.dot(pooled, wl_ref[...],
                               preferred_element_type=_f32) + bl_ref[...]


def _tcf_call(a128, dis, b3_128, bte, bto, wl, bl):
    full = pl.BlockSpec((_BT, LW), lambda i: (i, 0))
    return pl.pallas_call(
        _tcf_body,
        grid=(NP2 // _BT,),
        in_specs=[
            pl.BlockSpec((1, _BT, LW), lambda i: (0, i, 0)),
            pl.BlockSpec((1, _BT, LW), lambda i: (1, i, 0)),
            full,
            pl.BlockSpec((1, LW), lambda i: (0, 0)),
            pl.BlockSpec((1, _BT), lambda i: (0, i)),
            pl.BlockSpec((1, _BT), lambda i: (0, i)),
            pl.BlockSpec((H, OUT), lambda i: (0, 0)),
            pl.BlockSpec((1, OUT), lambda i: (0, 0)),
        ],
        out_specs=pl.BlockSpec((G, OUT), lambda i: (0, 0)),
        out_shape=jax.ShapeDtypeStruct((G, OUT), _f32),
        scratch_shapes=[
            pltpu.VMEM((G, H), _f32),
            pltpu.VMEM((G, 1), _f32),
        ],
    )(a128, a128, dis, b3_128, bte, bto, wl, bl)


def _blockdiag(w):
    z = jnp.zeros((H, H), _f32)
    return jnp.concatenate(
        [jnp.concatenate([w, z], axis=1),
         jnp.concatenate([z, w], axis=1)], axis=0)


def kernel(x, edge_index, batch, W1, b1, W2, b2, W3, b3, Wl, bl):
    pad_e = PE - E
    pad_src = jnp.arange(pad_e, dtype=jnp.int32) % N
    pad_dst = N + (jnp.arange(pad_e, dtype=jnp.int32) % (NPAD - N))
    src_p = jnp.concatenate([edge_index[0], pad_src]).reshape(EC, NW, CH)
    dst_p = jnp.concatenate([edge_index[1], pad_dst]).reshape(EC, NW, CH)
    x_p = jnp.pad(x, ((0, NPAD - N), (0, 0)))
    xe = x_p[0::2]
    xo = x_p[1::2]
    batch_p = jnp.concatenate(
        [batch, jnp.full((NPAD - N,), G, jnp.int32)])
    bte = batch_p[0::2].reshape(1, NP2)
    bto = batch_p[1::2].reshape(1, NP2)

    degs = _deg_kernel(dst_p)
    d0e = degs[0, 0::2].reshape(NP2, 1)
    d0o = degs[0, 1::2].reshape(NP2, 1)
    d1e = degs[1, 0::2].reshape(NP2, 1)
    d1o = degs[1, 1::2].reshape(NP2, 1)
    zh = jnp.zeros((F_IN, H), _f32)
    w1l = jnp.concatenate([W1, zh], axis=1)
    w1r = jnp.concatenate([zh, W1], axis=1)
    dis, y1 = _tc1_call(d0e, d1e, d0o, d1o, xe, xo, w1l, w1r)

    w2d = _blockdiag(W2)
    w3d = _blockdiag(W3)
    b1_128 = jnp.tile(b1, 2).reshape(1, LW)
    b2_128 = jnp.tile(b2, 2).reshape(1, LW)
    b3_128 = jnp.tile(b3, 2).reshape(1, LW)

    a = _mp_kernel(y1.reshape(NPAD, H), src_p, dst_p)
    y2 = _tc2_call(a.reshape(NC, NP2, LW), dis, b1_128, w2d)
    a = _mp_kernel(y2.reshape(NPAD, H), src_p, dst_p)
    y3 = _tc2_call(a.reshape(NC, NP2, LW), dis, b2_128, w3d)
    a = _mp_kernel(y3.reshape(NPAD, H), src_p, dst_p)
    return _tcf_call(a.reshape(NC, NP2, LW), dis, b3_128, bte, bto,
                     Wl, bl.reshape(1, OUT))
